# baseline (speedup 1.0000x reference)
import functools

import jax
import jax.numpy as jnp
from jax.experimental import pallas as pl
from jax.experimental.pallas import tpu as pltpu

E = 8
TOPK = 2
D = 768
N = 8192

T = 512


def _body(x_ref, W_ref, b_ref, Wb_ref, bb_ref, y_ref, idx_ref):
    x = x_ref[...]
    bids = jax.lax.dot_general(
        x, Wb_ref[...], (((1,), (1,)), ((), ())),
        preferred_element_type=jnp.float32,
    ) + bb_ref[...]

    ids = jax.lax.broadcasted_iota(jnp.int32, (T, E), 1)
    v1 = jnp.max(bids, axis=1, keepdims=True)
    i1 = jnp.min(jnp.where(bids == v1, ids, E), axis=1, keepdims=True)
    masked = jnp.where(ids == i1, -jnp.inf, bids)
    v2 = jnp.max(masked, axis=1, keepdims=True)
    i2 = jnp.min(jnp.where(masked == v2, ids, E), axis=1, keepdims=True)

    xb = x.astype(jnp.bfloat16)
    acc = jnp.zeros((T, D), jnp.float32)
    for e in range(E):
        sel = ((i1 == e) | (i2 == e)).astype(jnp.float32)
        out_e = jnp.dot(xb, W_ref[e], preferred_element_type=jnp.float32)
        acc = acc + sel * (out_e + b_ref[e][None, :])
    y_ref[...] = acc * 0.5
    idx_ref[...] = jnp.concatenate([i1, i2], axis=1)


@jax.jit
def kernel(x, W, b, Wb, bb):
    bb2 = bb.reshape(1, E)
    W = W.astype(jnp.bfloat16)
    grid = (N // T,)
    y, idx = pl.pallas_call(
        _body,
        grid=grid,
        in_specs=[
            pl.BlockSpec((T, D), lambda i: (i, 0)),
            pl.BlockSpec((E, D, D), lambda i: (0, 0, 0)),
            pl.BlockSpec((E, D), lambda i: (0, 0)),
            pl.BlockSpec((E, D), lambda i: (0, 0)),
            pl.BlockSpec((1, E), lambda i: (0, 0)),
        ],
        out_specs=[
            pl.BlockSpec((T, D), lambda i: (i, 0)),
            pl.BlockSpec((T, TOPK), lambda i: (i, 0)),
        ],
        out_shape=[
            jax.ShapeDtypeStruct((N, D), jnp.float32),
            jax.ShapeDtypeStruct((N, TOPK), jnp.int32),
        ],
    )(x, W, b, Wb, bb2)
    return y, idx


# speedup vs baseline: 1.0443x; 1.0443x over previous
import functools

import jax
import jax.numpy as jnp
from jax.experimental import pallas as pl
from jax.experimental.pallas import tpu as pltpu

E = 8
TOPK = 2
D = 768
N = 8192

T = 1024


def _body(x_ref, W_ref, b_ref, Wb_ref, bb_ref, y_ref, idx_ref):
    x = x_ref[...]
    bids = jax.lax.dot_general(
        x, Wb_ref[...], (((1,), (1,)), ((), ())),
        preferred_element_type=jnp.float32,
    ) + bb_ref[...]

    ids = jax.lax.broadcasted_iota(jnp.int32, (T, E), 1)
    v1 = jnp.max(bids, axis=1, keepdims=True)
    i1 = jnp.min(jnp.where(bids == v1, ids, E), axis=1, keepdims=True)
    masked = jnp.where(ids == i1, -jnp.inf, bids)
    v2 = jnp.max(masked, axis=1, keepdims=True)
    i2 = jnp.min(jnp.where(masked == v2, ids, E), axis=1, keepdims=True)

    acc = jnp.zeros((T, D), jnp.float32)
    for e in range(E):
        sel = ((i1 == e) | (i2 == e)).astype(jnp.float32)
        out_e = jnp.dot(x, W_ref[e], preferred_element_type=jnp.float32)
        acc = acc + sel * (out_e + b_ref[e][None, :])
    y_ref[...] = acc * 0.5
    idx_ref[...] = jnp.concatenate([i1, i2], axis=1)


@jax.jit
def kernel(x, W, b, Wb, bb):
    bb2 = bb.reshape(1, E)
    grid = (N // T,)
    y, idx = pl.pallas_call(
        _body,
        grid=grid,
        in_specs=[
            pl.BlockSpec((T, D), lambda i: (i, 0)),
            pl.BlockSpec((E, D, D), lambda i: (0, 0, 0)),
            pl.BlockSpec((E, D), lambda i: (0, 0)),
            pl.BlockSpec((E, D), lambda i: (0, 0)),
            pl.BlockSpec((1, E), lambda i: (0, 0)),
        ],
        out_specs=[
            pl.BlockSpec((T, D), lambda i: (i, 0)),
            pl.BlockSpec((T, TOPK), lambda i: (i, 0)),
        ],
        out_shape=[
            jax.ShapeDtypeStruct((N, D), jnp.float32),
            jax.ShapeDtypeStruct((N, TOPK), jnp.int32),
        ],
    )(x, W, b, Wb, bb2)
    return y, idx
